# baseline (speedup 1.0000x reference)
import jax
import jax.numpy as jnp
from jax.experimental import pallas as pl
from jax.experimental.pallas import tpu as pltpu
from jax.experimental.pallas import tpu_sc as plsc

D = 64
ROWS = 128
BR = 4
SCALE = 8.0
BI = 256


def kernel(x, table):
    B, L = x.shape
    N = B * L
    xi = x.reshape(N // ROWS, ROWS)

    mesh = plsc.VectorSubcoreMesh(core_axis_name="c", subcore_axis_name="s")

    @pl.kernel(
        out_type=jax.ShapeDtypeStruct((N, D), jnp.float32),
        mesh=mesh,
        compiler_params=pltpu.CompilerParams(use_tc_tiling_on_sc=False),
    )
    def gather_k(table_hbm, i_hbm, o_hbm):
        def body(i_vmem, o_vmem):
            for r in range(BR):
                pltpu.sync_copy(table_hbm.at[i_vmem.at[r]],
                                o_vmem.at[pl.ds(r * ROWS, ROWS)])

        pltpu.emit_pipeline(
            body,
            grid=(N // ROWS // BR,),
            in_specs=[pl.BlockSpec((BR, ROWS), index_map=lambda i: (i, 0))],
            out_specs=[pl.BlockSpec((BR * ROWS, D), index_map=lambda i: (i, 0))],
            core_axis_name=("c", "s"),
            dimension_semantics=(pltpu.PARALLEL,),
        )(i_hbm, o_hbm)

    raw = gather_k(table, xi)
    pairs = raw.reshape(N // 2, 128)

    def scale_k(p_ref, o_ref):
        v = p_ref[...] * SCALE
        ve = v[:, :D].reshape(BI, L // 2, 1, D)
        vo = v[:, D:].reshape(BI, L // 2, 1, D)
        o_ref[...] = jnp.concatenate([ve, vo], axis=2).reshape(BI, L, D)

    return pl.pallas_call(
        scale_k,
        grid=(B // BI,),
        in_specs=[pl.BlockSpec((BI * L * D // 128, 128), lambda i: (i, 0))],
        out_specs=pl.BlockSpec((BI, L, D), lambda i: (i, 0, 0)),
        out_shape=jax.ShapeDtypeStruct((B, L, D), jnp.float32),
    )(pairs)


# speedup vs baseline: 1.3894x; 1.3894x over previous
import jax
import jax.numpy as jnp
from jax.experimental import pallas as pl
from jax.experimental.pallas import tpu as pltpu
from jax.experimental.pallas import tpu_sc as plsc

D = 64
ROWS = 128
BR = 4
SCALE = 8.0


def kernel(x, table):
    B, L = x.shape
    N = B * L
    H = B // 2

    xi = x.T.reshape(L, 2, H).transpose(0, 2, 1).reshape(N // ROWS, ROWS)

    mesh = plsc.VectorSubcoreMesh(core_axis_name="c", subcore_axis_name="s")

    @pl.kernel(
        out_type=jax.ShapeDtypeStruct((N, D), jnp.float32),
        mesh=mesh,
        compiler_params=pltpu.CompilerParams(use_tc_tiling_on_sc=False),
    )
    def gather_k(table_hbm, i_hbm, o_hbm):
        def body(i_vmem, o_vmem):
            for r in range(BR):
                pltpu.sync_copy(table_hbm.at[i_vmem.at[r]],
                                o_vmem.at[pl.ds(r * ROWS, ROWS)])

        pltpu.emit_pipeline(
            body,
            grid=(N // ROWS // BR,),
            in_specs=[pl.BlockSpec((BR, ROWS), index_map=lambda i: (i, 0))],
            out_specs=[pl.BlockSpec((BR * ROWS, D), index_map=lambda i: (i, 0))],
            core_axis_name=("c", "s"),
            dimension_semantics=(pltpu.PARALLEL,),
        )(i_hbm, o_hbm)

    raw = gather_k(table, xi)
    pairs = raw.reshape(N // 2, 2 * D)

    CH = 1024

    def trans_k(p_ref, o_ref):
        for c in range(H // 2 // CH):
            v = p_ref[pl.ds(c * CH, CH), :] * SCALE
            t = v.T
            o_ref[0, :, pl.ds(c * CH, CH)] = t[:D]
            o_ref[0, :, pl.ds(H + c * CH, CH)] = t[D:]

    out = pl.pallas_call(
        trans_k,
        grid=(L,),
        in_specs=[pl.BlockSpec((H, 2 * D), lambda l: (l, 0))],
        out_specs=pl.BlockSpec((1, D, B), lambda l: (l, 0, 0)),
        out_shape=jax.ShapeDtypeStruct((L, D, B), jnp.float32),
        compiler_params=pltpu.CompilerParams(
            dimension_semantics=("parallel",)),
    )(pairs)

    return out.transpose(2, 0, 1)
